# baseline (speedup 1.0000x reference)
import functools

import jax
import jax.numpy as jnp
import numpy as np
from jax.experimental import pallas as pl
from jax.experimental.pallas import tpu as pltpu

_RATE = 0.5


@functools.lru_cache(maxsize=None)
def _feat_indices(b, hw, num):
    with jax.ensure_compile_time_eval():
        keys = jax.random.split(jax.random.key(42), b)
        fi = jax.vmap(lambda kk: jax.random.permutation(kk, hw)[:num])(keys)
        return np.asarray(fi)


def _lfm_kernel(sel_ref, sims_ref, fidx_ref, feat_ref, refs_ref,
                out_ref, ridx_ref, fused_acc, feat_sel_s, wfs_s, oh_feat_s):
    c, hw = feat_ref.shape[1], feat_ref.shape[2]
    num = fidx_ref.shape[2]
    n = pl.num_programs(1)
    bi = pl.program_id(0)
    ji = pl.program_id(1)

    iota = jax.lax.broadcasted_iota(jnp.int32, (hw, num), 0)

    @pl.when(ji == 0)
    def _feat_stage():
        feat = feat_ref[0]
        fidx = fidx_ref[0]
        oh_feat = (iota == fidx).astype(jnp.bfloat16)
        oh_feat_s[...] = oh_feat

        f_hi = feat.astype(jnp.bfloat16)
        r1 = feat - f_hi.astype(jnp.float32)
        f_mid = r1.astype(jnp.bfloat16)
        f_lo = (r1 - f_mid.astype(jnp.float32)).astype(jnp.bfloat16)
        dims_g = (((1,), (0,)), ((), ()))
        feat_sel = (jax.lax.dot_general(f_lo, oh_feat, dims_g,
                                        preferred_element_type=jnp.float32)
                    + jax.lax.dot_general(f_mid, oh_feat, dims_g,
                                          preferred_element_type=jnp.float32)
                    + jax.lax.dot_general(f_hi, oh_feat, dims_g,
                                          preferred_element_type=jnp.float32))
        feat_sel_s[...] = feat_sel
        norm_sel = jnp.maximum(
            jnp.sqrt(jnp.sum(feat_sel * feat_sel, axis=0, keepdims=True)),
            1e-12)
        w1 = feat_sel / norm_sel
        n2 = jnp.maximum(jnp.sqrt(jnp.sum(w1 * w1, axis=0, keepdims=True)),
                         1e-12)
        wfs_s[...] = (w1 / n2).astype(jnp.bfloat16)

        base_sim = sims_ref[bi, sel_ref[n]]
        fused_acc[...] = base_sim * feat_sel

    ref = refs_ref[0, 0].reshape(c, hw)
    rnorm = jnp.maximum(
        jnp.sqrt(jnp.sum(ref * ref, axis=0, keepdims=True)), 1e-12)
    wref = ref / rnorm
    fxT = jax.lax.dot_general(wref.astype(jnp.bfloat16), wfs_s[...],
                              (((0,), (0,)), ((), ())),
                              preferred_element_type=jnp.float32)
    cmax = jnp.max(fxT, axis=0, keepdims=True)
    amin = jnp.min(jnp.where(fxT == cmax, iota, hw), axis=0, keepdims=True)
    ridx_ref[0, 0] = amin
    oh = (iota == amin).astype(jnp.bfloat16)
    sj = sims_ref[bi, sel_ref[ji]]
    fused_acc[...] += sj * jax.lax.dot_general(
        ref.astype(jnp.bfloat16), oh, (((1,), (0,)), ((), ())),
        preferred_element_type=jnp.float32)

    @pl.when(ji == n - 1)
    def _out_stage():
        oh_feat = oh_feat_s[...]
        scat = jax.lax.dot_general(fused_acc[...].astype(jnp.bfloat16),
                                   oh_feat, (((1,), (1,)), ((), ())),
                                   preferred_element_type=jnp.float32)
        sel_mask = jax.lax.dot_general(jnp.ones((1, num), jnp.bfloat16),
                                       oh_feat, (((1,), (1,)), ((), ())),
                                       preferred_element_type=jnp.float32)
        out_ref[0] = jnp.where(sel_mask > 0.5, scat, feat_ref[0])


def kernel(feat, refs, index, similarity):
    b, k, c, h, w = refs.shape
    hw = h * w
    n = k - 1
    num = int(_RATE * hw)

    try:
        feat_indices = jnp.asarray(_feat_indices(b, hw, num))
    except Exception:
        keys = jax.random.split(jax.random.key(42), b)
        feat_indices = jax.vmap(
            lambda kk: jax.random.permutation(kk, hw)[:num])(keys)

    feat3 = feat.reshape(b, c, hw)
    sims = similarity.astype(jnp.float32)
    fidx3 = feat_indices.astype(jnp.int32).reshape(b, 1, num)
    pos = jnp.arange(n, dtype=jnp.int32)
    idx32 = jnp.asarray(index, jnp.int32)
    sel = jnp.concatenate(
        [jnp.where(pos < idx32, pos, pos + 1), idx32[None]])

    grid_spec = pltpu.PrefetchScalarGridSpec(
        num_scalar_prefetch=1,
        grid=(b, n),
        in_specs=[
            pl.BlockSpec(memory_space=pltpu.SMEM),
            pl.BlockSpec((1, 1, num), lambda i, j, sr: (i, 0, 0)),
            pl.BlockSpec((1, c, hw), lambda i, j, sr: (i, 0, 0)),
            pl.BlockSpec((1, 1, c, h, w), lambda i, j, sr: (i, sr[j], 0, 0, 0)),
        ],
        out_specs=[
            pl.BlockSpec((1, c, hw), lambda i, j, sr: (i, 0, 0)),
            pl.BlockSpec((1, 1, 1, num), lambda i, j, sr: (i, j, 0, 0)),
        ],
        scratch_shapes=[
            pltpu.VMEM((c, num), jnp.float32),
            pltpu.VMEM((c, num), jnp.float32),
            pltpu.VMEM((c, num), jnp.bfloat16),
            pltpu.VMEM((hw, num), jnp.bfloat16),
        ],
    )

    out3, ridx = pl.pallas_call(
        _lfm_kernel,
        grid_spec=grid_spec,
        out_shape=[
            jax.ShapeDtypeStruct((b, c, hw), jnp.float32),
            jax.ShapeDtypeStruct((b, n, 1, num), jnp.int32),
        ],
        compiler_params=pltpu.CompilerParams(
            dimension_semantics=("arbitrary", "arbitrary"),
        ),
    )(sel, sims, fidx3, feat3, refs)

    return out3.reshape(b, c, h, w), feat_indices, ridx.reshape(b, n, num)
